# Initial kernel scaffold; baseline (speedup 1.0000x reference)
import jax
import jax.numpy as jnp
from jax.experimental import pallas as pl


def kernel(h, x, edge_index, We1, be1, We2, be2, Wn1, bn1, Wn2, bn2, Wc1, bc1, Wc2, bc2):
    raise NotImplementedError("write your pallas kernel here")


# speedup vs baseline: 4.8058x; 4.8058x over previous
import functools

import jax
import jax.numpy as jnp
from jax import lax
from jax.experimental import pallas as pl
from jax.experimental.pallas import tpu as pltpu
from jax.experimental.pallas import tpu_sc as plsc

N = 10000
E = 320000
D = 128
HID = 128
STEPS = 32
GAMMA = 10.0
X_MIN = 0.0
X_MAX = 10.0

XW = 16
XC = 8
NC = 2
NS = 16
NW = NC * NS
CH = 128
NCHUNK = E // CH
NP = 10240
ROWS_PER_TILE = NP // NS

_f32 = jnp.float32


def _silu(z):
    return z / (1.0 + jnp.exp(-z))


def _pre_body(h_ref, w1a_ref, w1b_ref, be1_ref, a_ref, b_ref):
    h = h_ref[...]
    a_ref[...] = jnp.dot(h, w1a_ref[...], preferred_element_type=_f32)
    b_ref[...] = jnp.dot(h, w1b_ref[...], preferred_element_type=_f32) + be1_ref[...]


def _pre_call(h, w1a, w1b, be1, *, interpret=False):
    blk = 2000
    grid = (N // blk,)
    return pl.pallas_call(
        _pre_body,
        grid=grid,
        in_specs=[
            pl.BlockSpec((blk, D), lambda i: (i, 0)),
            pl.BlockSpec((D, HID), lambda i: (0, 0)),
            pl.BlockSpec((D, HID), lambda i: (0, 0)),
            pl.BlockSpec((1, HID), lambda i: (0, 0)),
        ],
        out_specs=[
            pl.BlockSpec((blk, HID), lambda i: (i, 0)),
            pl.BlockSpec((blk, HID), lambda i: (i, 0)),
        ],
        out_shape=[
            jax.ShapeDtypeStruct((N, HID), _f32),
            jax.ShapeDtypeStruct((N, HID), _f32),
        ],
        interpret=interpret,
    )(h, w1a, w1b, be1)


def _edge_body(ga_ref, gb_ref, xdt_ref, cent_ref, w1e_ref, we2_ref, be2_ref,
               wn1_ref, bn1_ref, wc1_ref, bc1_ref, wc2_ref, bc2_ref,
               u_ref, cut_ref):
    xd = xdt_ref[...].T
    d2 = xd[:, 3:4]
    d = jnp.sqrt(d2)
    e = jnp.exp(-GAMMA * (d - cent_ref[...]) ** 2)
    pre = (ga_ref[...] + gb_ref[...]
           + jnp.dot(e, w1e_ref[...], preferred_element_type=_f32))
    msg = jnp.dot(_silu(pre), we2_ref[...], preferred_element_type=_f32) + be2_ref[...]
    u_ref[...] = _silu(jnp.dot(msg, wn1_ref[...], preferred_element_type=_f32) + bn1_ref[...])
    cpre = _silu(jnp.dot(msg, wc1_ref[...], preferred_element_type=_f32) + bc1_ref[...])
    cmsg = jnp.dot(cpre, wc2_ref[...], preferred_element_type=_f32) + bc2_ref[...]
    f = cmsg / (d + 1.0)
    colid = lax.broadcasted_iota(jnp.int32, xd.shape, 1)
    cu = jnp.where(colid == 3, 1.0, jnp.where(colid > 3, 0.0, xd * f))
    cut_ref[...] = cu.T


def _edge_call(ga, gb, xdt, cent, w1e, we2, be2, wn1, bn1, wc1, bc1, wc2, bc2,
               *, interpret=False):
    blk = 1280
    grid = (E // blk,)
    full = lambda shape: pl.BlockSpec(shape, lambda i: tuple(0 for _ in shape))
    return pl.pallas_call(
        _edge_body,
        grid=grid,
        in_specs=[
            pl.BlockSpec((blk, HID), lambda i: (i, 0)),
            pl.BlockSpec((blk, HID), lambda i: (i, 0)),
            pl.BlockSpec((XC, blk), lambda i: (0, i)),
            full((1, STEPS)),
            full((STEPS, HID)),
            full((HID, HID)),
            full((1, HID)),
            full((HID, HID)),
            full((1, HID)),
            full((HID, HID)),
            full((1, HID)),
            full((HID, 1)),
            full((1, 1)),
        ],
        out_specs=[
            pl.BlockSpec((blk, HID), lambda i: (i, 0)),
            pl.BlockSpec((XC, blk), lambda i: (0, i)),
        ],
        out_shape=[
            jax.ShapeDtypeStruct((E, HID), _f32),
            jax.ShapeDtypeStruct((XC, E), _f32),
        ],
        interpret=interpret,
    )(ga, gb, xdt, cent, w1e, we2, be2, wn1, bn1, wc1, bc1, wc2, bc2)


def _post_body(h_ref, x_ref, au_ref, ac_ref,
               wn2_ref, bn2_ref, hp_ref, xp_ref):
    au = au_ref[...]
    ac = ac_ref[...]
    cnt = ac[:, 3:4]
    agg = jnp.dot(au, wn2_ref[...], preferred_element_type=_f32)
    mean = agg / jnp.clip(cnt, 1.0, None) + jnp.where(cnt > 0, 1.0, 0.0) * bn2_ref[...]
    hp_ref[...] = h_ref[...] + mean
    xp_ref[...] = x_ref[...] + ac[:, 0:3]


def _post_call(h, x, au, ac, wn2, bn2, *, interpret=False):
    blk = 2000
    grid = (N // blk,)
    return pl.pallas_call(
        _post_body,
        grid=grid,
        in_specs=[
            pl.BlockSpec((blk, D), lambda i: (i, 0)),
            pl.BlockSpec((blk, 3), lambda i: (i, 0)),
            pl.BlockSpec((blk, HID), lambda i: (i, 0)),
            pl.BlockSpec((blk, HID), lambda i: (i, 0)),
            pl.BlockSpec((HID, D), lambda i: (0, 0)),
            pl.BlockSpec((1, D), lambda i: (0, 0)),
        ],
        out_specs=[
            pl.BlockSpec((blk, D), lambda i: (i, 0)),
            pl.BlockSpec((blk, 3), lambda i: (i, 0)),
        ],
        out_shape=[
            jax.ShapeDtypeStruct((N, D), _f32),
            jax.ShapeDtypeStruct((N, 3), _f32),
        ],
        interpret=interpret,
    )(h, x, au, ac, wn2, bn2)


def _sc_mesh():
    return plsc.VectorSubcoreMesh(
        core_axis_name="c", subcore_axis_name="s", num_cores=NC, num_subcores=NS)


def _gather_sc_body(a_hbm, b_hbm, x4_hbm, src_hbm, tar_hbm,
                    ga_hbm, gb_hbm, xdt_hbm,
                    srcv, tarv, gav, gbv, x4v, xdtv, sem):
    wid = lax.axis_index("s") * NC + lax.axis_index("c")
    nck = 78 + jnp.where(wid < NCHUNK - 78 * NW, 1, 0)

    pltpu.sync_copy(x4_hbm, x4v)
    zeros16 = jnp.zeros((16,), _f32)
    for r in range(3, XC):
        for g in range(CH // 16):
            xdtv[r, pl.ds(g * 16, 16)] = zeros16

    def chunk(i, carry):
        off = (wid + i * NW) * CH
        pltpu.sync_copy(src_hbm.at[pl.ds(off, CH)], srcv)
        pltpu.sync_copy(tar_hbm.at[pl.ds(off, CH)], tarv)
        cp1 = pltpu.async_copy(a_hbm.at[srcv], gav, sem)
        cp2 = pltpu.async_copy(b_hbm.at[tarv], gbv, sem)
        for g in range(CH // 16):
            s16 = srcv[pl.ds(g * 16, 16)] * 4
            t16 = tarv[pl.ds(g * 16, 16)] * 4
            dx = plsc.load_gather(x4v, [s16]) - plsc.load_gather(x4v, [t16])
            dy = plsc.load_gather(x4v, [s16 + 1]) - plsc.load_gather(x4v, [t16 + 1])
            dz = plsc.load_gather(x4v, [s16 + 2]) - plsc.load_gather(x4v, [t16 + 2])
            xdtv[0, pl.ds(g * 16, 16)] = dx
            xdtv[1, pl.ds(g * 16, 16)] = dy
            xdtv[2, pl.ds(g * 16, 16)] = dz
            xdtv[3, pl.ds(g * 16, 16)] = dx * dx + dy * dy + dz * dz
        cp1.wait()
        cp2.wait()
        pltpu.sync_copy(gav, ga_hbm.at[pl.ds(off, CH)])
        pltpu.sync_copy(gbv, gb_hbm.at[pl.ds(off, CH)])
        pltpu.sync_copy(xdtv, xdt_hbm.at[:, pl.ds(off, CH)])
        return carry

    lax.fori_loop(0, nck, chunk, 0)


def _gather_call(a, b, x4flat, src, tar):
    kern = pl.kernel(
        _gather_sc_body,
        out_type=(
            jax.ShapeDtypeStruct((E, HID), _f32),
            jax.ShapeDtypeStruct((E, HID), _f32),
            jax.ShapeDtypeStruct((XC, E), _f32),
        ),
        mesh=_sc_mesh(),
        scratch_types=(
            pltpu.VMEM((CH,), jnp.int32),
            pltpu.VMEM((CH,), jnp.int32),
            pltpu.VMEM((CH, HID), _f32),
            pltpu.VMEM((CH, HID), _f32),
            pltpu.VMEM((4 * N,), _f32),
            pltpu.VMEM((XC, CH), _f32),
            pltpu.SemaphoreType.DMA,
        ),
        compiler_params=pltpu.CompilerParams(needs_layout_passes=False),
    )
    return kern(a, b, x4flat, src, tar)


def _scatter_sc_body(tar_hbm, u_hbm, cut_hbm, z_hbm, rows_hbm,
                     au_hbm, ac_hbm,
                     tarv, uv, cutv, ridv, acc_sh, sem):
    cid = lax.axis_index("c")
    sid = lax.axis_index("s")
    nck = (NCHUNK // NS) + jnp.where(sid < NCHUNK - (NCHUNK // NS) * NS, 1, 0)
    rows0 = sid * ROWS_PER_TILE
    lanes16 = lax.iota(jnp.int32, 16)

    def zblk(k, carry):
        r = rows0 + k * CH
        pltpu.sync_copy(rows_hbm.at[pl.ds(r, CH)], ridv)
        pltpu.sync_copy(z_hbm.at[pl.ds(r, CH)], uv)
        pltpu.sync_copy(uv, acc_sh.at[ridv])
        return carry

    lax.fori_loop(0, ROWS_PER_TILE // CH, zblk, 0)
    plsc.subcore_barrier()

    def chunk_u(i, carry):
        off = (sid + i * NS) * CH
        pltpu.sync_copy(tar_hbm.at[pl.ds(off, CH)], tarv)
        pltpu.sync_copy(u_hbm.at[pl.ds(off, CH)], uv)
        pltpu.sync_copy(uv, acc_sh.at[tarv], add=True)
        return carry

    def chunk_c(i, carry):
        off = (sid + i * NS) * CH
        pltpu.sync_copy(tar_hbm.at[pl.ds(off, CH)], tarv)
        pltpu.sync_copy(cut_hbm.at[:, pl.ds(off, CH)], cutv)
        for g in range(CH // 16):
            rows16 = lanes16 + g * 16
            for comp in range(4):
                v = cutv[comp, pl.ds(g * 16, 16)]
                plsc.store_scatter(
                    uv, [rows16, jnp.full((16,), comp, jnp.int32)], v)
        pltpu.sync_copy(uv, acc_sh.at[tarv], add=True)
        return carry

    @pl.when(cid == 0)
    def _():
        lax.fori_loop(0, nck, chunk_u, 0)

    @pl.when(cid == 1)
    def _():
        lax.fori_loop(0, nck, chunk_c, 0)

    plsc.subcore_barrier()

    def dblk(k, carry):
        r = rows0 + k * CH
        pltpu.sync_copy(rows_hbm.at[pl.ds(r, CH)], ridv)
        pltpu.sync_copy(acc_sh.at[ridv], uv)

        @pl.when(cid == 0)
        def _():
            pltpu.sync_copy(uv, au_hbm.at[pl.ds(r, CH)])

        @pl.when(cid == 1)
        def _():
            pltpu.sync_copy(uv, ac_hbm.at[pl.ds(r, CH)])

        return carry

    lax.fori_loop(0, ROWS_PER_TILE // CH, dblk, 0)


def _scatter_call(tar, u, cut, zeros_n, rowids):
    kern = pl.kernel(
        _scatter_sc_body,
        out_type=(
            jax.ShapeDtypeStruct((NP, HID), _f32),
            jax.ShapeDtypeStruct((NP, HID), _f32),
        ),
        mesh=_sc_mesh(),
        scratch_types=(
            pltpu.VMEM((CH,), jnp.int32),
            pltpu.VMEM((CH, HID), _f32),
            pltpu.VMEM((XC, CH), _f32),
            pltpu.VMEM((CH,), jnp.int32),
            pltpu.VMEM_SHARED((NP, HID), _f32),
            pltpu.SemaphoreType.DMA,
        ),
        compiler_params=pltpu.CompilerParams(needs_layout_passes=False),
    )
    return kern(tar, u, cut, zeros_n, rowids)


def kernel(h, x, edge_index, We1, be1, We2, be2, Wn1, bn1, Wn2, bn2,
           Wc1, bc1, Wc2, bc2):
    src = edge_index[0]
    tar = edge_index[1]
    w1a = We1[:D]
    w1b = We1[D:2 * D]
    w1e = We1[2 * D:]
    x4flat = jnp.concatenate([x, jnp.zeros((N, 1), _f32)], axis=1).reshape(4 * N)
    cent = jnp.linspace(X_MIN, X_MAX, STEPS, dtype=_f32).reshape(1, STEPS)
    zeros_n = jnp.zeros((NP, HID), _f32)
    rowids = jnp.arange(NP, dtype=jnp.int32)

    a, b = _pre_call(h, w1a, w1b, be1.reshape(1, HID))
    ga, gb, xdt = _gather_call(a, b, x4flat, src, tar)
    u, cut = _edge_call(ga, gb, xdt, cent, w1e, We2, be2.reshape(1, HID),
                        Wn1, bn1.reshape(1, HID), Wc1, bc1.reshape(1, HID),
                        Wc2, bc2.reshape(1, 1))
    au, ac = _scatter_call(tar, u, cut, zeros_n, rowids)
    hp, xp = _post_call(h, x, au[:N], ac[:N], Wn2, bn2.reshape(1, D))
    return hp, xp
